# Initial kernel scaffold; baseline (speedup 1.0000x reference)
import jax
import jax.numpy as jnp
from jax.experimental import pallas as pl


def kernel(x, edge_index, W, b, W_hi, b_hi, W_lo, b_lo):
    raise NotImplementedError("write your pallas kernel here")


# speedup vs baseline: 27.7921x; 27.7921x over previous
import functools

import jax
import jax.numpy as jnp
from jax import lax
from jax.experimental import pallas as pl
from jax.experimental.pallas import tpu as pltpu
from jax.experimental.pallas import tpu_sc as plsc

N = 10000
D = 128
NC = 2
NS = 16
NW = NC * NS
WIN = 128
NPAD = 10240
HROWS = NPAD // 128
ROWS_PER_TILE = NPAD // NS

_mesh = plsc.VectorSubcoreMesh(core_axis_name="c", subcore_axis_name="s")


def _deg_body(nwin, col2d_hbm, zeros_hbm, out_hbm, colbuf, onesbuf, acc):
    c = lax.axis_index("c")
    s = lax.axis_index("s")
    wid = c * NS + s

    @pl.loop(0, WIN)
    def _(j):
        onesbuf[j, pl.ds(0, 16)] = jnp.ones((16,), jnp.float32)

    pltpu.sync_copy(
        zeros_hbm.at[pl.ds(s * ROWS_PER_TILE, ROWS_PER_TILE)],
        acc.at[pl.ds(s * ROWS_PER_TILE, ROWS_PER_TILE)],
    )
    pltpu.sync_copy(col2d_hbm.at[pl.ds(wid * nwin, nwin)], colbuf)
    plsc.subcore_barrier()

    @pl.loop(0, nwin)
    def _(j):
        pltpu.sync_copy(onesbuf, acc.at[colbuf.at[j]], add=True)

    plsc.subcore_barrier()
    pltpu.sync_copy(
        acc.at[pl.ds(s * ROWS_PER_TILE, ROWS_PER_TILE)],
        out_hbm.at[c, pl.ds(s * ROWS_PER_TILE, ROWS_PER_TILE)],
    )


def _scatter_body(nwin, row2d_hbm, col2d_hbm, hs_hbm, out_hbm,
                  rowbuf, colbuf, gbuf, sem, acc):
    c = lax.axis_index("c")
    s = lax.axis_index("s")
    wid = c * NS + s

    pltpu.sync_copy(
        hs_hbm.at[pl.ds(s * ROWS_PER_TILE, ROWS_PER_TILE)],
        acc.at[pl.ds(s * ROWS_PER_TILE, ROWS_PER_TILE)],
    )
    pltpu.sync_copy(row2d_hbm.at[pl.ds(wid * nwin, nwin)], rowbuf)
    pltpu.sync_copy(col2d_hbm.at[pl.ds(wid * nwin, nwin)], colbuf)
    plsc.subcore_barrier()

    @pl.loop(0, nwin)
    def _(j):
        pltpu.async_copy(hs_hbm.at[rowbuf.at[j]], gbuf, sem).wait()
        pltpu.sync_copy(gbuf, acc.at[colbuf.at[j]], add=True)

    plsc.subcore_barrier()
    pltpu.sync_copy(
        acc.at[pl.ds(s * ROWS_PER_TILE, ROWS_PER_TILE)],
        out_hbm.at[c, pl.ds(s * ROWS_PER_TILE, ROWS_PER_TILE)],
    )


def _hs_body(x_ref, W_ref, Wh_ref, d0_ref, d1_ref, hs_ref):
    Wc = jnp.dot(W_ref[...], Wh_ref[...], preferred_element_type=jnp.float32)
    dinv = lax.rsqrt(d0_ref[...] + d1_ref[...] + 1.0)
    hs_ref[...] = (
        jnp.dot(x_ref[...], Wc, preferred_element_type=jnp.float32) * dinv
    )


def _final_body(accp_ref, hs_ref, d0_ref, d1_ref, b2_ref, Wh_ref, bh_ref,
                hi_ref, lo_ref):
    dinv = lax.rsqrt(d0_ref[...] + d1_ref[...] + 1.0)
    bc = jnp.dot(b2_ref[...], Wh_ref[...],
                 preferred_element_type=jnp.float32) + bh_ref[...]
    out2 = (accp_ref[0] + accp_ref[1] - hs_ref[...]) * dinv + bc
    hi_ref[...] = out2[:N, :64]
    lo_ref[...] = out2[:N, 64:]


def kernel(x, edge_index, W, b, W_hi, b_hi, W_lo, b_lo):
    E = edge_index.shape[1]
    nwin = -(-E // (NW * WIN))
    nwin = -(-nwin // 8) * 8
    e_pad = NW * nwin * WIN
    pad = e_pad - E

    row = edge_index[0]
    col = edge_index[1]
    pidx = jnp.arange(pad, dtype=jnp.int32)
    row_p = jnp.concatenate([row, (pidx * 37) % N]).reshape(e_pad // WIN, WIN)
    col_p = jnp.concatenate([col, N + (pidx % 64)]).reshape(e_pad // WIN, WIN)
    xp = jnp.concatenate([x, jnp.zeros((NPAD - N, D), x.dtype)])

    deg_partials = pl.kernel(
        functools.partial(_deg_body, nwin),
        out_type=jax.ShapeDtypeStruct((NC, NPAD, 16), jnp.float32),
        mesh=_mesh,
        scratch_types=[
            pltpu.VMEM((nwin, WIN), jnp.int32),
            pltpu.VMEM((WIN, 16), jnp.float32),
            pltpu.VMEM_SHARED((NPAD, 16), jnp.float32),
        ],
    )(col_p, jnp.zeros((NPAD, 16), jnp.float32))

    d0 = deg_partials[0, :, :1]
    d1 = deg_partials[1, :, :1]
    Wh = jnp.concatenate([W_hi, W_lo], axis=1)
    bh = jnp.concatenate([b_hi, b_lo])[None, :]
    b2 = b[None, :]

    hs = pl.pallas_call(
        _hs_body,
        out_shape=jax.ShapeDtypeStruct((NPAD, D), jnp.float32),
    )(xp, W, Wh, d0, d1)

    acc_partials = pl.kernel(
        functools.partial(_scatter_body, nwin),
        out_type=jax.ShapeDtypeStruct((NC, NPAD, D), jnp.float32),
        mesh=_mesh,
        scratch_types=[
            pltpu.VMEM((nwin, WIN), jnp.int32),
            pltpu.VMEM((nwin, WIN), jnp.int32),
            pltpu.VMEM((WIN, D), jnp.float32),
            pltpu.SemaphoreType.DMA,
            pltpu.VMEM_SHARED((NPAD, D), jnp.float32),
        ],
    )(row_p, col_p, hs)

    hi, lo = pl.pallas_call(
        _final_body,
        out_shape=[
            jax.ShapeDtypeStruct((N, 64), jnp.float32),
            jax.ShapeDtypeStruct((N, 64), jnp.float32),
        ],
    )(acc_partials, hs, d0, d1, b2, Wh, bh)
    return (hi, lo)


# speedup vs baseline: 31.1643x; 1.1213x over previous
import functools

import jax
import jax.numpy as jnp
from jax import lax
from jax.experimental import pallas as pl
from jax.experimental.pallas import tpu as pltpu
from jax.experimental.pallas import tpu_sc as plsc

N = 10000
D = 128
NC = 2
NS = 16
NW = NC * NS
WIN = 128
NPAD = 10240
HROWS = NPAD // 128
ROWS_PER_TILE = NPAD // NS

_mesh = plsc.VectorSubcoreMesh(core_axis_name="c", subcore_axis_name="s")


def _deg_body(nwin, col2d_hbm, zeros_hbm, out_hbm, colbuf, onesbuf, sem, acc):
    c = lax.axis_index("c")
    s = lax.axis_index("s")
    wid = c * NS + s

    @pl.loop(0, WIN)
    def _(j):
        onesbuf[j, pl.ds(0, 16)] = jnp.ones((16,), jnp.float32)

    pltpu.sync_copy(
        zeros_hbm.at[pl.ds(s * ROWS_PER_TILE, ROWS_PER_TILE)],
        acc.at[pl.ds(s * ROWS_PER_TILE, ROWS_PER_TILE)],
    )
    pltpu.sync_copy(col2d_hbm.at[pl.ds(wid * nwin, nwin)], colbuf)
    plsc.subcore_barrier()

    @pl.loop(0, nwin)
    def _(j):
        pltpu.sync_copy(onesbuf, acc.at[colbuf.at[j]], add=True)

    plsc.subcore_barrier()
    pltpu.sync_copy(
        acc.at[pl.ds(s * ROWS_PER_TILE, ROWS_PER_TILE)],
        out_hbm.at[c, pl.ds(s * ROWS_PER_TILE, ROWS_PER_TILE)],
    )


CHUNK = 40


def _scatter_body(nwin, row2d_hbm, col2d_hbm, hs_hbm, out_hbm,
                  rowchunk, colchunk, gbufa, gbufb,
                  gsem0, gsem1, ssem0, ssem1, acc):
    c = lax.axis_index("c")
    s = lax.axis_index("s")
    wid = c * NS + s
    gbuf = (gbufa, gbufb)
    gsem = (gsem0, gsem1)
    ssem = (ssem0, ssem1)

    pltpu.sync_copy(
        hs_hbm.at[pl.ds(s * ROWS_PER_TILE, ROWS_PER_TILE)],
        acc.at[pl.ds(s * ROWS_PER_TILE, ROWS_PER_TILE)],
    )
    plsc.subcore_barrier()

    for j0 in range(0, 80, CHUNK):
        pltpu.sync_copy(row2d_hbm.at[pl.ds(wid * nwin + j0, CHUNK)],
                        rowchunk)
        pltpu.sync_copy(col2d_hbm.at[pl.ds(wid * nwin + j0, CHUNK)],
                        colchunk)
        for p in range(2):
            pltpu.async_copy(hs_hbm.at[rowchunk.at[p]], gbuf[p], gsem[p])

        @pl.loop(0, CHUNK, step=2)
        def _(jj):
            for p in range(2):
                pltpu.make_async_copy(
                    hs_hbm.at[rowchunk.at[jj + p]], gbuf[p], gsem[p]).wait()
                pltpu.async_copy(
                    gbuf[p], acc.at[colchunk.at[jj + p]], ssem[p], add=True)
            for p in range(2):
                pltpu.make_async_copy(
                    gbuf[p], acc.at[colchunk.at[jj + p]], ssem[p]).wait()

                @pl.when(jj + p + 2 < CHUNK)
                def _():
                    pltpu.async_copy(
                        hs_hbm.at[rowchunk.at[jj + p + 2]], gbuf[p], gsem[p])

    plsc.subcore_barrier()
    pltpu.sync_copy(
        acc.at[pl.ds(s * ROWS_PER_TILE, ROWS_PER_TILE)],
        out_hbm.at[c, pl.ds(s * ROWS_PER_TILE, ROWS_PER_TILE)],
    )


def _hs_body(x_ref, W_ref, Wh_ref, d0_ref, d1_ref, hs_ref):
    Wc = jnp.dot(W_ref[...], Wh_ref[...], preferred_element_type=jnp.float32)
    dinv = lax.rsqrt(d0_ref[...] + d1_ref[...] + 1.0)
    hs_ref[...] = (
        jnp.dot(x_ref[...], Wc, preferred_element_type=jnp.float32) * dinv
    )


def _final_body(accp_ref, hs_ref, d0_ref, d1_ref, b2_ref, Wh_ref, bh_ref,
                hi_ref, lo_ref):
    dinv = lax.rsqrt(d0_ref[...] + d1_ref[...] + 1.0)
    bc = jnp.dot(b2_ref[...], Wh_ref[...],
                 preferred_element_type=jnp.float32) + bh_ref[...]
    out2 = (accp_ref[0] + accp_ref[1] - hs_ref[...]) * dinv + bc
    hi_ref[...] = out2[:N, :64]
    lo_ref[...] = out2[:N, 64:]


def kernel(x, edge_index, W, b, W_hi, b_hi, W_lo, b_lo):
    E = edge_index.shape[1]
    nwin = -(-E // (NW * WIN))
    nwin = -(-nwin // 8) * 8
    e_pad = NW * nwin * WIN
    pad = e_pad - E

    row = edge_index[0]
    col = edge_index[1]
    pidx = jnp.arange(pad, dtype=jnp.int32)
    row_p = jnp.concatenate([row, (pidx * 37) % N]).reshape(e_pad // WIN, WIN)
    col_p = jnp.concatenate([col, N + (pidx % 64)]).reshape(e_pad // WIN, WIN)
    xp = jnp.concatenate([x, jnp.zeros((NPAD - N, D), x.dtype)])

    deg_partials = pl.kernel(
        functools.partial(_deg_body, nwin),
        out_type=jax.ShapeDtypeStruct((NC, NPAD, 16), jnp.float32),
        mesh=_mesh,
        scratch_types=[
            pltpu.VMEM((nwin, WIN), jnp.int32),
            pltpu.VMEM((WIN, 16), jnp.float32),
            pltpu.SemaphoreType.DMA,
            pltpu.VMEM_SHARED((NPAD, 16), jnp.float32),
        ],
    )(col_p, jnp.zeros((NPAD, 16), jnp.float32))

    d0 = deg_partials[0, :, :1]
    d1 = deg_partials[1, :, :1]
    Wh = jnp.concatenate([W_hi, W_lo], axis=1)
    bh = jnp.concatenate([b_hi, b_lo])[None, :]
    b2 = b[None, :]

    hs = pl.pallas_call(
        _hs_body,
        out_shape=jax.ShapeDtypeStruct((NPAD, D), jnp.float32),
    )(xp, W, Wh, d0, d1)

    acc_partials = pl.kernel(
        functools.partial(_scatter_body, nwin),
        out_type=jax.ShapeDtypeStruct((NC, NPAD, D), jnp.float32),
        mesh=_mesh,
        scratch_types=[
            pltpu.VMEM((CHUNK, WIN), jnp.int32),
            pltpu.VMEM((CHUNK, WIN), jnp.int32),
            pltpu.VMEM((WIN, D), jnp.float32),
            pltpu.VMEM((WIN, D), jnp.float32),
            pltpu.SemaphoreType.DMA,
            pltpu.SemaphoreType.DMA,
            pltpu.SemaphoreType.DMA,
            pltpu.SemaphoreType.DMA,
            pltpu.VMEM_SHARED((NPAD, D), jnp.float32),
        ],
    )(row_p, col_p, hs)

    hi, lo = pl.pallas_call(
        _final_body,
        out_shape=[
            jax.ShapeDtypeStruct((N, 64), jnp.float32),
            jax.ShapeDtypeStruct((N, 64), jnp.float32),
        ],
    )(acc_partials, hs, d0, d1, b2, Wh, bh)
    return (hi, lo)
